# baseline (speedup 1.0000x reference)
import jax
import jax.numpy as jnp
from jax import lax
from jax.experimental import pallas as pl
from jax.experimental.pallas import tpu as pltpu
from jax.experimental.pallas import tpu_sc as plsc

_ROWS = 4880
_DIM = 128
_HEAD = 3664
_TAIL = _ROWS - _HEAD
_TAIL_WORDS = _TAIL * _DIM
_NUM_CORES = 2
_NUM_SUBCORES = 16
_NW = _NUM_CORES * _NUM_SUBCORES
_CHUNK = _TAIL_WORDS // _NW


def _sc_body(src_hbm, out_hbm, buf):
    wid = lax.axis_index("s") * _NUM_CORES + lax.axis_index("c")
    base = wid * _CHUNK
    pltpu.sync_copy(src_hbm.at[pl.ds(_HEAD * _DIM + base, _CHUNK)], buf)
    pltpu.sync_copy(buf, out_hbm.at[pl.ds(base, _CHUNK)])


def _tc_body(head_ref, tail_ref, out_ref):
    out_ref[0:_HEAD, :] = head_ref[...]
    out_ref[_HEAD:_ROWS, :] = tail_ref[...]


@jax.jit
def kernel(table):
    flat = table.reshape(_ROWS * _DIM)
    mesh = plsc.VectorSubcoreMesh(core_axis_name="c", subcore_axis_name="s")
    sc_out = pl.kernel(
        _sc_body,
        out_type=jax.ShapeDtypeStruct((_TAIL_WORDS,), jnp.float32),
        scratch_types=[pltpu.VMEM((_CHUNK,), jnp.float32)],
        mesh=mesh,
    )(flat)
    return pl.pallas_call(
        _tc_body,
        out_shape=jax.ShapeDtypeStruct((_ROWS, _DIM), jnp.float32),
        grid=(1,),
        in_specs=[
            pl.BlockSpec((_HEAD, _DIM), lambda i: (0, 0)),
            pl.BlockSpec((_TAIL, _DIM), lambda i: (0, 0)),
        ],
        out_specs=pl.BlockSpec((_ROWS, _DIM), lambda i: (0, 0)),
    )(table, sc_out.reshape(_TAIL, _DIM))


# speedup vs baseline: 1.1173x; 1.1173x over previous
import jax
import jax.numpy as jnp
from jax import lax
from jax.experimental import pallas as pl
from jax.experimental.pallas import tpu as pltpu
from jax.experimental.pallas import tpu_sc as plsc

_ROWS = 4880
_DIM = 128
_TOTAL = _ROWS * _DIM
_NUM_SUBCORES = 16
_CHUNK = _TOTAL // _NUM_SUBCORES


def _copy_body(src_hbm, out_hbm, buf):
    wid = lax.axis_index("s")
    base = wid * _CHUNK
    pltpu.sync_copy(src_hbm.at[pl.ds(base, _CHUNK)], buf)
    pltpu.sync_copy(buf, out_hbm.at[pl.ds(base, _CHUNK)])


@jax.jit
def kernel(table):
    flat = table.reshape(_TOTAL)
    mesh = plsc.VectorSubcoreMesh(
        core_axis_name="c", subcore_axis_name="s", num_cores=1)
    out = pl.kernel(
        _copy_body,
        out_type=jax.ShapeDtypeStruct((_TOTAL,), jnp.float32),
        scratch_types=[pltpu.VMEM((_CHUNK,), jnp.float32)],
        mesh=mesh,
    )(flat)
    return out.reshape(_ROWS, _DIM)
